# Initial kernel scaffold; baseline (speedup 1.0000x reference)
import jax
import jax.numpy as jnp
from jax.experimental import pallas as pl


def kernel(x, position_ids, inv_freq):
    raise NotImplementedError("write your pallas kernel here")


# speedup vs baseline: 1.7668x; 1.7668x over previous
import functools

import jax
import jax.numpy as jnp
from jax import lax
from jax.experimental import pallas as pl
from jax.experimental.pallas import tpu as pltpu
from jax.experimental.pallas import tpu_sc as plsc

SEQ = 16384
D = 64
V = 4096

NC = 2
NS = 16
NW = NC * NS
BPW = SEQ // NW
CHUNK = 128
NCHUNK = BPW // CHUNK


def _tables_body(inv_ref, tab_ref):
    f = inv_ref[...]
    tab_ref[...] = jnp.concatenate([jnp.cos(f), jnp.sin(f)], axis=-1)


def _build_table(inv_freq):
    return pl.pallas_call(
        _tables_body,
        out_shape=jax.ShapeDtypeStruct((V, 2 * D), jnp.float32),
    )(inv_freq)


def _sc_gather_body(tab, idx_hbm, out_hbm, idx_v, buf, sem):
    wid = lax.axis_index("s") * NC + lax.axis_index("c")
    pltpu.sync_copy(idx_hbm.at[wid], idx_v)
    copies = []
    for j in range(NCHUNK):
        copies.append(pltpu.async_copy(tab.at[idx_v.at[j]], buf.at[j], sem))
    for c in copies:
        c.wait()
    pltpu.sync_copy(buf, out_hbm.at[wid])


_sc_gather = functools.partial(
    pl.kernel,
    out_type=jax.ShapeDtypeStruct((NW, NCHUNK, CHUNK, 2 * D), jnp.float32),
    mesh=plsc.VectorSubcoreMesh(
        core_axis_name="c", subcore_axis_name="s",
        num_cores=NC, num_subcores=NS,
    ),
    scratch_types=[
        pltpu.VMEM((NCHUNK, CHUNK), jnp.int32),
        pltpu.VMEM((NCHUNK, CHUNK, 2 * D), jnp.float32),
        pltpu.SemaphoreType.DMA,
    ],
)(_sc_gather_body)


def kernel(x, position_ids, inv_freq):
    tab = _build_table(inv_freq)
    idx = position_ids.reshape(NW, NCHUNK, CHUNK)
    packed = _sc_gather(tab, idx).reshape(SEQ, 2 * D)
    cos = packed[:, :D].reshape(1, SEQ, D).astype(x.dtype)
    sin = packed[:, D:].reshape(1, SEQ, D).astype(x.dtype)
    return (cos, sin)
